# baseline (speedup 1.0000x reference)
import jax
import jax.numpy as jnp
from jax.experimental import pallas as pl
from jax.experimental.pallas import tpu as pltpu

_N = 2048
_IN = 16384
_K = 40
_BLK = 128
_NB = _N // _BLK


def _fused_body(inp_ref, conn_ref, out_ref, ov_blk, ov_rep, part):
    s = pl.program_id(0)
    c = pl.program_id(1)
    ov_half = jnp.sum(conn_ref[:] * inp_ref[:], axis=1)

    @pl.when(c == 0)
    def _first_half():
        part[:] = ov_half.reshape(1, _BLK)

    @pl.when(c == 1)
    def _second_half():
        ov = part[0, :] + ov_half
        ov_blk[pl.ds(s, 1), :] = ov.reshape(1, _BLK)
        ov_rep[pl.ds(s * _BLK, _BLK), :] = jnp.broadcast_to(
            ov.reshape(_BLK, 1), (_BLK, _BLK)
        )

    @pl.when((s == _NB - 1) & (c == 1))
    def _rank_and_mask():
        planes = ov_blk[:]
        tri = (
            jax.lax.broadcasted_iota(jnp.int32, (_BLK, _BLK), 1)
            < jax.lax.broadcasted_iota(jnp.int32, (_BLK, _BLK), 0)
        )
        for b in range(_NB):
            lo, hi = b * _BLK, (b + 1) * _BLK
            ocr = ov_rep[lo:hi, :]
            gt3 = planes[b:, None, :] > ocr[None, :, :]
            rank = jnp.sum(jnp.where(gt3, 1.0, 0.0), axis=(0, 2))
            if b > 0:
                geq3 = planes[:b, None, :] >= ocr[None, :, :]
                rank = rank + jnp.sum(jnp.where(geq3, 1.0, 0.0), axis=(0, 2))
            eq_dg = (planes[b:b + 1, :] == ocr) & tri
            rank = rank + jnp.sum(jnp.where(eq_dg, 1.0, 0.0), axis=1)
            out_ref[b:b + 1, :] = (rank < float(_K)).astype(
                jnp.float32
            ).reshape(1, _BLK)


def kernel(input_vector, connections):
    mask = pl.pallas_call(
        _fused_body,
        grid=(_NB, 2),
        in_specs=[
            pl.BlockSpec((1, _IN // 2), lambda i, c: (0, c)),
            pl.BlockSpec((_BLK, _IN // 2), lambda i, c: (i, c)),
        ],
        out_specs=pl.BlockSpec((_NB, _BLK), lambda i, c: (0, 0)),
        out_shape=jax.ShapeDtypeStruct((_NB, _BLK), jnp.float32),
        scratch_shapes=[
            pltpu.VMEM((_NB, _BLK), jnp.float32),
            pltpu.VMEM((_N, _BLK), jnp.float32),
            pltpu.VMEM((1, _BLK), jnp.float32),
        ],
    )(input_vector.reshape(1, _IN), connections)
    return mask.reshape(_N)


# speedup vs baseline: 1.2065x; 1.2065x over previous
import jax
import jax.numpy as jnp
from jax.experimental import pallas as pl
from jax.experimental.pallas import tpu as pltpu

_N = 2048
_IN = 16384
_K = 40
_BLK = 128
_NB = _N // _BLK


def _fused_body(inp_ref, conn_ref, out_ref, ov_blk, ov_rep):
    s = pl.program_id(0)
    ov = jnp.sum(conn_ref[:] * inp_ref[:], axis=1)
    ov_blk[pl.ds(s, 1), :] = ov.reshape(1, _BLK)
    ov_rep[pl.ds(s * _BLK, _BLK), :] = jnp.broadcast_to(
        ov.reshape(_BLK, 1), (_BLK, _BLK)
    )

    @pl.when(s == _NB - 1)
    def _rank_and_mask():
        planes = ov_blk[:]
        tri = (
            jax.lax.broadcasted_iota(jnp.int32, (_BLK, _BLK), 1)
            < jax.lax.broadcasted_iota(jnp.int32, (_BLK, _BLK), 0)
        )
        for b in range(_NB):
            lo, hi = b * _BLK, (b + 1) * _BLK
            ocr = ov_rep[lo:hi, :]
            gt3 = planes[b:, None, :] > ocr[None, :, :]
            rank = jnp.sum(jnp.where(gt3, 1.0, 0.0), axis=(0, 2))
            if b > 0:
                geq3 = planes[:b, None, :] >= ocr[None, :, :]
                rank = rank + jnp.sum(jnp.where(geq3, 1.0, 0.0), axis=(0, 2))
            eq_dg = (planes[b:b + 1, :] == ocr) & tri
            rank = rank + jnp.sum(jnp.where(eq_dg, 1.0, 0.0), axis=1)
            out_ref[b:b + 1, :] = (rank < float(_K)).astype(
                jnp.float32
            ).reshape(1, _BLK)


def kernel(input_vector, connections):
    mask = pl.pallas_call(
        _fused_body,
        grid=(_NB,),
        in_specs=[
            pl.BlockSpec((1, _IN), lambda i: (0, 0)),
            pl.BlockSpec((_BLK, _IN), lambda i: (i, 0)),
        ],
        out_specs=pl.BlockSpec((_NB, _BLK), lambda i: (0, 0)),
        out_shape=jax.ShapeDtypeStruct((_NB, _BLK), jnp.float32),
        scratch_shapes=[
            pltpu.VMEM((_NB, _BLK), jnp.float32),
            pltpu.VMEM((_N, _BLK), jnp.float32),
        ],
    )(input_vector.reshape(1, _IN), connections)
    return mask.reshape(_N)
